# baseline (speedup 1.0000x reference)
import math

import jax
import jax.numpy as jnp
from jax import lax
from jax.experimental import pallas as pl
from jax.experimental.pallas import tpu as pltpu
from jax.experimental.pallas import tpu_sc as plsc

N = 50000
E = 800000
IN_DIM = 128
H = 64
HH = 32
ED = 16
EMB = 512
G = 64

NC = 2
NS = 16
LANES = 16

CH = 128
EP = 800768
EPH = EP // 2
SCAN_CH = 1024
SCAN_NCH = EPH // SCAN_CH
BUCKET = 3200
CAPH = 28672
EA_ROWS = NC * NS * CAPH
TROWS = 3208
LTRASH = BUCKET
NROWS = NS * BUCKET
TRASH = N

_BN_SCALE = 1.0 / math.sqrt(1.0 + 1e-5)


def _nodeproj_body(x_ref, w_ref, b_ref, out_ref):
    r = jnp.dot(x_ref[...], w_ref[...], preferred_element_type=jnp.float32)
    r = r + b_ref[...]
    out_ref[0] = r[:, :HH]
    out_ref[1] = r[:, HH:]


def _node_proj(x, np_W, np_b):
    B = 2000
    nb = N // B
    return pl.pallas_call(
        _nodeproj_body,
        grid=(nb,),
        in_specs=[
            pl.BlockSpec((B, IN_DIM), lambda i: (i, 0)),
            pl.BlockSpec((IN_DIM, H), lambda i: (0, 0)),
            pl.BlockSpec((1, H), lambda i: (0, 0)),
        ],
        out_specs=pl.BlockSpec((2, B, HH), lambda i: (0, i, 0)),
        out_shape=jax.ShapeDtypeStruct((2, N, HH), jnp.float32),
    )(x, np_W, np_b.reshape(1, H))


def _edgelin_body(ea_ref, w_ref, b_ref, o1_ref, o2_ref, o3_ref):
    r = jnp.dot(ea_ref[...], w_ref[...], preferred_element_type=jnp.float32)
    r = r + b_ref[...]
    o1_ref[0] = r[:, 0:32]
    o1_ref[1] = r[:, 32:64]
    o2_ref[0] = r[:, 64:96]
    o2_ref[1] = r[:, 96:128]
    o3_ref[0] = r[:, 128:160]
    o3_ref[1] = r[:, 160:192]


def _edge_lin(ea_bucketed, w_all, b_all):
    B = 2048
    nb = EA_ROWS // B
    out_sds = jax.ShapeDtypeStruct((2, EA_ROWS, HH), jnp.float32)
    spec = pl.BlockSpec((2, B, HH), lambda i: (0, i, 0))
    return pl.pallas_call(
        _edgelin_body,
        grid=(nb,),
        in_specs=[
            pl.BlockSpec((B, ED), lambda i: (i, 0)),
            pl.BlockSpec((ED, 3 * H), lambda i: (0, 0)),
            pl.BlockSpec((1, 3 * H), lambda i: (0, 0)),
        ],
        out_specs=(spec, spec, spec),
        out_shape=(out_sds, out_sds, out_sds),
    )(ea_bucketed, w_all, b_all.reshape(1, 3 * H))


def _nodemlp_body(h_ref, a_ref, w1_ref, b1_ref, w2_ref, b2_ref, g_ref, bb_ref,
                  out_ref):
    hf = jnp.concatenate([h_ref[0], h_ref[1]], axis=1)
    af = jnp.concatenate([a_ref[0], a_ref[1]], axis=1)
    t = hf + af
    t = jnp.maximum(
        jnp.dot(t, w1_ref[...], preferred_element_type=jnp.float32)
        + b1_ref[...], 0.0)
    t = jnp.dot(t, w2_ref[...], preferred_element_type=jnp.float32) + b2_ref[...]
    t = t * (g_ref[...] * _BN_SCALE) + bb_ref[...]
    t = jnp.maximum(t, 0.0)
    out_ref[0] = t[:, :HH]
    out_ref[1] = t[:, HH:]


def _node_mlp(h2, aggr2, W1, b1, W2, b2, bn_g, bn_b):
    B = 2000
    nb = N // B
    spec = pl.BlockSpec((2, B, HH), lambda i: (0, i, 0))
    vec = lambda v: v.reshape(1, H)
    return pl.pallas_call(
        _nodemlp_body,
        grid=(nb,),
        in_specs=[
            spec, spec,
            pl.BlockSpec((H, H), lambda i: (0, 0)),
            pl.BlockSpec((1, H), lambda i: (0, 0)),
            pl.BlockSpec((H, H), lambda i: (0, 0)),
            pl.BlockSpec((1, H), lambda i: (0, 0)),
            pl.BlockSpec((1, H), lambda i: (0, 0)),
            pl.BlockSpec((1, H), lambda i: (0, 0)),
        ],
        out_specs=spec,
        out_shape=jax.ShapeDtypeStruct((2, N, HH), jnp.float32),
    )(h2, aggr2, W1, vec(b1), W2, vec(b2), vec(bn_g), vec(bn_b))


def _pool_body(h_ref, batch_ref, pw_ref, pb_ref, lg_ref, lb_ref, out_ref,
               acc_ref, mx_ref):
    i = pl.program_id(0)
    nb = pl.num_programs(0)

    @pl.when(i == 0)
    def _init():
        acc_ref[...] = jnp.zeros_like(acc_ref)
        mx_ref[...] = jnp.full_like(mx_ref, -jnp.inf)

    hf = jnp.concatenate([h_ref[0], h_ref[1]], axis=1)
    B = hf.shape[0]
    bb = batch_ref[0, 0]
    gid = lax.broadcasted_iota(jnp.int32, (1, G), 1)
    onehot = (bb[:, None] == gid).astype(jnp.float32)
    ones = jnp.ones((B, 1), jnp.float32)
    hx = jnp.concatenate([hf, ones, jnp.zeros((B, 63), jnp.float32)], axis=1)
    acc_ref[...] += jnp.dot(onehot.T, hx, preferred_element_type=jnp.float32)

    bbc = bb[:, None]
    parts = []
    for g in range(G):
        col = jnp.where(bbc == g, hf, -jnp.inf)
        parts.append(jnp.max(col, axis=0, keepdims=True))
    mx_ref[...] = jnp.maximum(mx_ref[...], jnp.concatenate(parts, axis=0))

    @pl.when(i == nb - 1)
    def _final():
        acc = acc_ref[...]
        sums = acc[:, :H]
        cnt = acc[:, H:H + 1]
        mean = sums / jnp.maximum(cnt, 1.0)
        cat = jnp.concatenate([mean, mx_ref[...]], axis=1)
        o = jnp.dot(cat, pw_ref[...], preferred_element_type=jnp.float32)
        o = o + pb_ref[...]
        mu = jnp.mean(o, axis=-1, keepdims=True)
        var = jnp.mean((o - mu) * (o - mu), axis=-1, keepdims=True)
        o = (o - mu) / jnp.sqrt(var + 1e-5) * lg_ref[...] + lb_ref[...]
        out_ref[...] = jnp.maximum(o, 0.0)


def _pool_proj(h2, batch, proj_W, proj_b, ln_g, ln_b):
    B = 1000
    nb = N // B
    batch_r = batch.reshape(nb, 1, B)
    return pl.pallas_call(
        _pool_body,
        grid=(nb,),
        in_specs=[
            pl.BlockSpec((2, B, HH), lambda i: (0, i, 0)),
            pl.BlockSpec((1, 1, B), lambda i: (i, 0, 0)),
            pl.BlockSpec((2 * H, EMB), lambda i: (0, 0)),
            pl.BlockSpec((1, EMB), lambda i: (0, 0)),
            pl.BlockSpec((1, EMB), lambda i: (0, 0)),
            pl.BlockSpec((1, EMB), lambda i: (0, 0)),
        ],
        out_specs=pl.BlockSpec((G, EMB), lambda i: (0, 0)),
        out_shape=jax.ShapeDtypeStruct((G, EMB), jnp.float32),
        scratch_shapes=[
            pltpu.VMEM((G, 2 * H), jnp.float32),
            pltpu.VMEM((G, H), jnp.float32),
        ],
    )(h2, batch_r, proj_W, proj_b.reshape(1, EMB), ln_g.reshape(1, EMB),
      ln_b.reshape(1, EMB))


def _part_body(src_hbm, dst_hbm, ea_hbm, pk_hbm, eab_hbm, cnt_hbm,
               sbuf, dbuf, pkbuf, eidbuf, earows, cntv, sem):
    c = lax.axis_index("c")
    s = lax.axis_index("s")
    iot = lax.broadcasted_iota(jnp.int32, (LANES,), 0)
    half_base = c * EPH
    s3200 = s * BUCKET

    def _scan_chunk(t, offv):
        base = half_base + t * SCAN_CH
        pltpu.sync_copy(src_hbm.at[pl.ds(base, SCAN_CH)], sbuf)
        pltpu.sync_copy(dst_hbm.at[pl.ds(base, SCAN_CH)], dbuf)

        def _vec(v, _):
            d = dbuf[pl.ds(v * LANES, LANES)]
            sv = sbuf[pl.ds(v * LANES, LANES)]
            b = lax.shift_right_logical(
                lax.shift_right_logical(d, 6) * 1311, 16)
            m = b == s
            mi = jnp.where(m, jnp.full((LANES,), 1, jnp.int32),
                           jnp.zeros((LANES,), jnp.int32))
            dloc = d - s3200
            pk = sv | lax.shift_left(dloc, 16)
            offv = cntv[...]
            pos = offv + plsc.cumsum(mi) - mi
            plsc.store_scatter(pkbuf, [pos], pk, mask=m)
            eid = (base + v * LANES) + iot
            plsc.store_scatter(eidbuf, [pos], eid, mask=m)
            cntv[...] = offv + plsc.all_reduce_population_count(m)
            return 0

        return lax.fori_loop(0, SCAN_CH // LANES, _vec, 0)

    cntv[...] = jnp.zeros((LANES,), jnp.int32)
    lax.fori_loop(0, SCAN_NCH, _scan_chunk, 0)
    offv = cntv[...]

    trash_pk = jnp.full((LANES,), LTRASH << 16, jnp.int32)
    zero16 = jnp.zeros((LANES,), jnp.int32)
    for j in range(CH // LANES):
        pos = offv + iot + j * LANES
        plsc.store_scatter(pkbuf, [pos], trash_pk)
        plsc.store_scatter(eidbuf, [pos], zero16)

    row = c * NS + s
    pltpu.sync_copy(cntv, cnt_hbm.at[pl.ds(row * LANES, LANES)])
    pltpu.sync_copy(pkbuf, pk_hbm.at[pl.ds(row * CAPH, CAPH)])

    n = jnp.max(offv)
    nch = lax.shift_right_logical(n + (CH - 1), 7)

    def _gather_chunk(t, _):
        gat = pltpu.async_copy(
            ea_hbm.at[eidbuf.at[pl.ds(t * CH, CH)]], earows, sem)
        gat.wait()
        pltpu.sync_copy(earows, eab_hbm.at[pl.ds(row * CAPH + t * CH, CH)])
        return 0

    lax.fori_loop(0, nch, _gather_chunk, 0)


def _make_sc_partition():
    mesh = plsc.VectorSubcoreMesh(core_axis_name="c", subcore_axis_name="s")
    return pl.kernel(
        _part_body,
        out_type=(
            jax.ShapeDtypeStruct((NC * NS * CAPH,), jnp.int32),
            jax.ShapeDtypeStruct((EA_ROWS, ED), jnp.float32),
            jax.ShapeDtypeStruct((NC * NS * LANES,), jnp.int32),
        ),
        mesh=mesh,
        scratch_types=[
            pltpu.VMEM((SCAN_CH,), jnp.int32),
            pltpu.VMEM((SCAN_CH,), jnp.int32),
            pltpu.VMEM((CAPH,), jnp.int32),
            pltpu.VMEM((CAPH,), jnp.int32),
            pltpu.VMEM((CH, ED), jnp.float32),
            pltpu.VMEM((LANES,), jnp.int32),
            pltpu.SemaphoreType.DMA,
        ],
        compiler_params=pltpu.CompilerParams(use_tc_tiling_on_sc=False, needs_layout_passes=False),
    )


_sc_partition = _make_sc_partition()


def _agg_body(h_hbm, e_hbm, pk_hbm, cnt_hbm, out_hbm,
              pkbuf, sidx, dloc, hrows, erows, cntv, aggr, sem):
    c = lax.axis_index("c")
    s = lax.axis_index("s")
    iot = lax.broadcasted_iota(jnp.int32, (LANES,), 0)
    zf = jnp.zeros((LANES,), jnp.float32)
    cN = c * N

    def _zrow(r, _):
        aggr[r, pl.ds(0, LANES)] = zf
        aggr[r, pl.ds(LANES, LANES)] = zf
        return 0
    lax.fori_loop(0, TROWS, _zrow, 0)

    for g in range(NC):
        row = g * NS + s
        pltpu.sync_copy(cnt_hbm.at[pl.ds(row * LANES, LANES)], cntv)
        n = jnp.max(cntv[...])
        nch = lax.shift_right_logical(n + (CH - 1), 7)
        pkbase = row * CAPH
        ebase = c * EA_ROWS + row * CAPH

        def _chunk(t, _):
            pltpu.sync_copy(pk_hbm.at[pl.ds(pkbase + t * CH, CH)], pkbuf)
            for v in range(CH // LANES):
                pkv = pkbuf[pl.ds(v * LANES, LANES)]
                sidx[pl.ds(v * LANES, LANES)] = (pkv & 0xFFFF) + cN
                dloc[pl.ds(v * LANES, LANES)] = \
                    lax.shift_right_logical(pkv, 16)
            gat = pltpu.async_copy(h_hbm.at[sidx], hrows, sem)
            pltpu.sync_copy(e_hbm.at[pl.ds(ebase + t * CH, CH)], erows)
            gat.wait()
            for v in range(CH // LANES):
                jv = iot + v * LANES
                dl = dloc[pl.ds(v * LANES, LANES)]
                for k in range(HH):
                    kv = jnp.full((LANES,), k, jnp.int32)
                    hv = plsc.load_gather(hrows, [jv, kv])
                    ev = plsc.load_gather(erows, [jv, kv])
                    val = jnp.maximum(hv + ev, 0.0)
                    plsc.addupdate_scatter(aggr, [dl, kv], val)
            return 0

        lax.fori_loop(0, nch, _chunk, 0)

    pltpu.sync_copy(aggr.at[pl.ds(0, BUCKET)],
                    out_hbm.at[pl.ds(c * NROWS + s * BUCKET, BUCKET)])


def _make_sc_aggr():
    mesh = plsc.VectorSubcoreMesh(core_axis_name="c", subcore_axis_name="s")
    return pl.kernel(
        _agg_body,
        out_type=jax.ShapeDtypeStruct((NC * NROWS, HH), jnp.float32),
        mesh=mesh,
        scratch_types=[
            pltpu.VMEM((CH,), jnp.int32),
            pltpu.VMEM((CH,), jnp.int32),
            pltpu.VMEM((CH,), jnp.int32),
            pltpu.VMEM((CH, HH), jnp.float32),
            pltpu.VMEM((CH, HH), jnp.float32),
            pltpu.VMEM((LANES,), jnp.int32),
            pltpu.VMEM((TROWS, HH), jnp.float32),
            pltpu.SemaphoreType.DMA,
        ],
        compiler_params=pltpu.CompilerParams(use_tc_tiling_on_sc=False, needs_layout_passes=False),
    )


_sc_aggr = _make_sc_aggr()


def _sc_layer(h2, eB, pkB, counts):
    h_flat = h2.reshape(2 * N, HH)
    e_flat = eB.reshape(2 * EA_ROWS, HH)
    out = _sc_aggr(h_flat, e_flat, pkB, counts)
    return out.reshape(2, NROWS, HH)[:, :N, :]


@jax.jit
def kernel(x, edge_index, edge_attr, batch, np_W, np_b,
           lin1_W, lin1_b, mlp1_W1, mlp1_b1, mlp1_W2, mlp1_b2, bn1_g, bn1_b,
           lin2_W, lin2_b, mlp2_W1, mlp2_b1, mlp2_W2, mlp2_b2, bn2_g, bn2_b,
           lin3_W, lin3_b, mlp3_W1, mlp3_b1, mlp3_W2, mlp3_b2, bn3_g, bn3_b,
           proj_W, proj_b, ln_g, ln_b):
    src = edge_index[0]
    dst = edge_index[1]
    srcp = jnp.pad(src, (0, EP - E))
    dstp = jnp.pad(dst, (0, EP - E), constant_values=TRASH)
    edge_attr_p = jnp.pad(edge_attr, ((0, EP - E), (0, 0)))

    pkB, ea_bucketed, counts = _sc_partition(srcp, dstp, edge_attr_p)

    w_all = jnp.concatenate([lin1_W, lin2_W, lin3_W], axis=1)
    b_all = jnp.concatenate([lin1_b, lin2_b, lin3_b], axis=0)
    e1, e2, e3 = _edge_lin(ea_bucketed, w_all, b_all)

    h = _node_proj(x, np_W, np_b)

    aggr = _sc_layer(h, e1, pkB, counts)
    h = _node_mlp(h, aggr, mlp1_W1, mlp1_b1, mlp1_W2, mlp1_b2, bn1_g, bn1_b)

    aggr = _sc_layer(h, e2, pkB, counts)
    h = _node_mlp(h, aggr, mlp2_W1, mlp2_b1, mlp2_W2, mlp2_b2, bn2_g, bn2_b)

    aggr = _sc_layer(h, e3, pkB, counts)
    h = _node_mlp(h, aggr, mlp3_W1, mlp3_b1, mlp3_W2, mlp3_b2, bn3_g, bn3_b)

    return _pool_proj(h, batch, proj_W, proj_b, ln_g, ln_b)


# speedup vs baseline: 1.7838x; 1.7838x over previous
import math

import jax
import jax.numpy as jnp
from jax import lax
from jax.experimental import pallas as pl
from jax.experimental.pallas import tpu as pltpu
from jax.experimental.pallas import tpu_sc as plsc

N = 50000
E = 800000
IN_DIM = 128
H = 64
HH = 32
ED = 16
EMB = 512
G = 64

NC = 2
NS = 16
LANES = 16

CH = 128
EP = 800768
EPH = EP // 2
SCAN_CH = 1024
SCAN_NCH = EPH // SCAN_CH
BUCKET = 3200
CAPH = 28672
EA_ROWS = NC * NS * CAPH
TROWS = 3208
LTRASH = BUCKET
NROWS = NS * BUCKET
TRASH = N

_BN_SCALE = 1.0 / math.sqrt(1.0 + 1e-5)


def _nodeproj_body(x_ref, w_ref, b_ref, out_ref):
    r = jnp.dot(x_ref[...], w_ref[...], preferred_element_type=jnp.float32)
    r = r + b_ref[...]
    out_ref[0] = r[:, :HH]
    out_ref[1] = r[:, HH:]


def _node_proj(x, np_W, np_b):
    B = 2000
    nb = N // B
    return pl.pallas_call(
        _nodeproj_body,
        grid=(nb,),
        in_specs=[
            pl.BlockSpec((B, IN_DIM), lambda i: (i, 0)),
            pl.BlockSpec((IN_DIM, H), lambda i: (0, 0)),
            pl.BlockSpec((1, H), lambda i: (0, 0)),
        ],
        out_specs=pl.BlockSpec((2, B, HH), lambda i: (0, i, 0)),
        out_shape=jax.ShapeDtypeStruct((2, N, HH), jnp.float32),
    )(x, np_W, np_b.reshape(1, H))


def _edgelin_body(ea_ref, w_ref, b_ref, o1_ref, o2_ref, o3_ref):
    r = jnp.dot(ea_ref[...], w_ref[...], preferred_element_type=jnp.float32)
    r = r + b_ref[...]
    o1_ref[0] = r[:, 0:32]
    o1_ref[1] = r[:, 32:64]
    o2_ref[0] = r[:, 64:96]
    o2_ref[1] = r[:, 96:128]
    o3_ref[0] = r[:, 128:160]
    o3_ref[1] = r[:, 160:192]


def _edge_lin(ea_bucketed, w_all, b_all):
    B = 2048
    nb = EA_ROWS // B
    out_sds = jax.ShapeDtypeStruct((2, EA_ROWS, HH), jnp.float32)
    spec = pl.BlockSpec((2, B, HH), lambda i: (0, i, 0))
    return pl.pallas_call(
        _edgelin_body,
        grid=(nb,),
        in_specs=[
            pl.BlockSpec((B, ED), lambda i: (i, 0)),
            pl.BlockSpec((ED, 3 * H), lambda i: (0, 0)),
            pl.BlockSpec((1, 3 * H), lambda i: (0, 0)),
        ],
        out_specs=(spec, spec, spec),
        out_shape=(out_sds, out_sds, out_sds),
    )(ea_bucketed, w_all, b_all.reshape(1, 3 * H))


def _nodemlp_body(h_ref, a_ref, w1_ref, b1_ref, w2_ref, b2_ref, g_ref, bb_ref,
                  out_ref):
    hf = jnp.concatenate([h_ref[0], h_ref[1]], axis=1)
    af = jnp.concatenate([a_ref[0], a_ref[1]], axis=1)
    t = hf + af
    t = jnp.maximum(
        jnp.dot(t, w1_ref[...], preferred_element_type=jnp.float32)
        + b1_ref[...], 0.0)
    t = jnp.dot(t, w2_ref[...], preferred_element_type=jnp.float32) + b2_ref[...]
    t = t * (g_ref[...] * _BN_SCALE) + bb_ref[...]
    t = jnp.maximum(t, 0.0)
    out_ref[0] = t[:, :HH]
    out_ref[1] = t[:, HH:]


def _node_mlp(h2, aggr2, W1, b1, W2, b2, bn_g, bn_b):
    B = 2000
    nb = N // B
    spec = pl.BlockSpec((2, B, HH), lambda i: (0, i, 0))
    vec = lambda v: v.reshape(1, H)
    return pl.pallas_call(
        _nodemlp_body,
        grid=(nb,),
        in_specs=[
            spec, spec,
            pl.BlockSpec((H, H), lambda i: (0, 0)),
            pl.BlockSpec((1, H), lambda i: (0, 0)),
            pl.BlockSpec((H, H), lambda i: (0, 0)),
            pl.BlockSpec((1, H), lambda i: (0, 0)),
            pl.BlockSpec((1, H), lambda i: (0, 0)),
            pl.BlockSpec((1, H), lambda i: (0, 0)),
        ],
        out_specs=spec,
        out_shape=jax.ShapeDtypeStruct((2, N, HH), jnp.float32),
    )(h2, aggr2, W1, vec(b1), W2, vec(b2), vec(bn_g), vec(bn_b))


def _pool_body(h_ref, batch_ref, pw_ref, pb_ref, lg_ref, lb_ref, out_ref,
               acc_ref, mx_ref):
    i = pl.program_id(0)
    nb = pl.num_programs(0)

    @pl.when(i == 0)
    def _init():
        acc_ref[...] = jnp.zeros_like(acc_ref)
        mx_ref[...] = jnp.full_like(mx_ref, -jnp.inf)

    hf = jnp.concatenate([h_ref[0], h_ref[1]], axis=1)
    B = hf.shape[0]
    bb = batch_ref[0, 0]
    gid = lax.broadcasted_iota(jnp.int32, (1, G), 1)
    onehot = (bb[:, None] == gid).astype(jnp.float32)
    ones = jnp.ones((B, 1), jnp.float32)
    hx = jnp.concatenate([hf, ones, jnp.zeros((B, 63), jnp.float32)], axis=1)
    acc_ref[...] += jnp.dot(onehot.T, hx, preferred_element_type=jnp.float32)

    bbc = bb[:, None]
    parts = []
    for g in range(G):
        col = jnp.where(bbc == g, hf, -jnp.inf)
        parts.append(jnp.max(col, axis=0, keepdims=True))
    mx_ref[...] = jnp.maximum(mx_ref[...], jnp.concatenate(parts, axis=0))

    @pl.when(i == nb - 1)
    def _final():
        acc = acc_ref[...]
        sums = acc[:, :H]
        cnt = acc[:, H:H + 1]
        mean = sums / jnp.maximum(cnt, 1.0)
        cat = jnp.concatenate([mean, mx_ref[...]], axis=1)
        o = jnp.dot(cat, pw_ref[...], preferred_element_type=jnp.float32)
        o = o + pb_ref[...]
        mu = jnp.mean(o, axis=-1, keepdims=True)
        var = jnp.mean((o - mu) * (o - mu), axis=-1, keepdims=True)
        o = (o - mu) / jnp.sqrt(var + 1e-5) * lg_ref[...] + lb_ref[...]
        out_ref[...] = jnp.maximum(o, 0.0)


def _pool_proj(h2, batch, proj_W, proj_b, ln_g, ln_b):
    B = 1000
    nb = N // B
    batch_r = batch.reshape(nb, 1, B)
    return pl.pallas_call(
        _pool_body,
        grid=(nb,),
        in_specs=[
            pl.BlockSpec((2, B, HH), lambda i: (0, i, 0)),
            pl.BlockSpec((1, 1, B), lambda i: (i, 0, 0)),
            pl.BlockSpec((2 * H, EMB), lambda i: (0, 0)),
            pl.BlockSpec((1, EMB), lambda i: (0, 0)),
            pl.BlockSpec((1, EMB), lambda i: (0, 0)),
            pl.BlockSpec((1, EMB), lambda i: (0, 0)),
        ],
        out_specs=pl.BlockSpec((G, EMB), lambda i: (0, 0)),
        out_shape=jax.ShapeDtypeStruct((G, EMB), jnp.float32),
        scratch_shapes=[
            pltpu.VMEM((G, 2 * H), jnp.float32),
            pltpu.VMEM((G, H), jnp.float32),
        ],
    )(h2, batch_r, proj_W, proj_b.reshape(1, EMB), ln_g.reshape(1, EMB),
      ln_b.reshape(1, EMB))


def _part_body(src_hbm, dst_hbm, ea_hbm, pk_hbm, eab_hbm, cnt_hbm,
               sbuf, dbuf, pkbuf, eidbuf, earows, cntv, sem):
    c = lax.axis_index("c")
    s = lax.axis_index("s")
    iot = lax.broadcasted_iota(jnp.int32, (LANES,), 0)
    half_base = c * EPH
    s3200 = s * BUCKET

    def _scan_chunk(t, offv):
        base = half_base + t * SCAN_CH
        pltpu.sync_copy(src_hbm.at[pl.ds(base, SCAN_CH)], sbuf)
        pltpu.sync_copy(dst_hbm.at[pl.ds(base, SCAN_CH)], dbuf)

        def _vec(v, _):
            d = dbuf[pl.ds(v * LANES, LANES)]
            sv = sbuf[pl.ds(v * LANES, LANES)]
            b = lax.shift_right_logical(
                lax.shift_right_logical(d, 6) * 1311, 16)
            m = b == s
            mi = jnp.where(m, jnp.full((LANES,), 1, jnp.int32),
                           jnp.zeros((LANES,), jnp.int32))
            dloc = d - s3200
            pk = sv | lax.shift_left(dloc, 16)
            offv = cntv[...]
            pos = offv + plsc.cumsum(mi) - mi
            plsc.store_scatter(pkbuf, [pos], pk, mask=m)
            eid = (base + v * LANES) + iot
            plsc.store_scatter(eidbuf, [pos], eid, mask=m)
            cntv[...] = offv + plsc.all_reduce_population_count(m)
            return 0

        return lax.fori_loop(0, SCAN_CH // LANES, _vec, 0)

    cntv[...] = jnp.zeros((LANES,), jnp.int32)
    lax.fori_loop(0, SCAN_NCH, _scan_chunk, 0)
    offv = cntv[...]

    trash_pk = jnp.full((LANES,), LTRASH << 16, jnp.int32)
    zero16 = jnp.zeros((LANES,), jnp.int32)
    for j in range(CH // LANES):
        pos = offv + iot + j * LANES
        plsc.store_scatter(pkbuf, [pos], trash_pk)
        plsc.store_scatter(eidbuf, [pos], zero16)

    row = c * NS + s
    pltpu.sync_copy(cntv, cnt_hbm.at[pl.ds(row * LANES, LANES)])
    pltpu.sync_copy(pkbuf, pk_hbm.at[pl.ds(row * CAPH, CAPH)])

    n = jnp.max(offv)
    nch = lax.shift_right_logical(n + (CH - 1), 7)

    def _gather_chunk(t, _):
        gat = pltpu.async_copy(
            ea_hbm.at[eidbuf.at[pl.ds(t * CH, CH)]], earows, sem)
        gat.wait()
        pltpu.sync_copy(earows, eab_hbm.at[pl.ds(row * CAPH + t * CH, CH)])
        return 0

    lax.fori_loop(0, nch, _gather_chunk, 0)


def _make_sc_partition():
    mesh = plsc.VectorSubcoreMesh(core_axis_name="c", subcore_axis_name="s")
    return pl.kernel(
        _part_body,
        out_type=(
            jax.ShapeDtypeStruct((NC * NS * CAPH,), jnp.int32),
            jax.ShapeDtypeStruct((EA_ROWS, ED), jnp.float32),
            jax.ShapeDtypeStruct((NC * NS * LANES,), jnp.int32),
        ),
        mesh=mesh,
        scratch_types=[
            pltpu.VMEM((SCAN_CH,), jnp.int32),
            pltpu.VMEM((SCAN_CH,), jnp.int32),
            pltpu.VMEM((CAPH,), jnp.int32),
            pltpu.VMEM((CAPH,), jnp.int32),
            pltpu.VMEM((CH, ED), jnp.float32),
            pltpu.VMEM((LANES,), jnp.int32),
            pltpu.SemaphoreType.DMA,
        ],
        compiler_params=pltpu.CompilerParams(use_tc_tiling_on_sc=False, needs_layout_passes=False),
    )


_sc_partition = _make_sc_partition()


def _agg_body(h_hbm, e_hbm, pk_hbm, cnt_hbm, out_hbm,
              pkbuf, sidx, dloc, hrows, erows, cntv, aggr, sem):
    c = lax.axis_index("c")
    s = lax.axis_index("s")
    iot = lax.broadcasted_iota(jnp.int32, (LANES,), 0)
    iot16 = iot + LANES
    jsel = [jnp.full((LANES,), j, jnp.int32) for j in range(LANES)]
    zf = jnp.zeros((LANES,), jnp.float32)
    cN = c * N

    def _zrow(r, _):
        aggr[r, pl.ds(0, LANES)] = zf
        aggr[r, pl.ds(LANES, LANES)] = zf
        return 0
    lax.fori_loop(0, TROWS, _zrow, 0)

    for g in range(NC):
        row = g * NS + s
        pltpu.sync_copy(cnt_hbm.at[pl.ds(row * LANES, LANES)], cntv)
        n = jnp.max(cntv[...])
        nch = lax.shift_right_logical(n + (CH - 1), 7)
        pkbase = row * CAPH
        ebase = c * EA_ROWS + row * CAPH

        def _chunk(t, _):
            pltpu.sync_copy(pk_hbm.at[pl.ds(pkbase + t * CH, CH)], pkbuf)
            for v in range(CH // LANES):
                pkv = pkbuf[pl.ds(v * LANES, LANES)]
                sidx[pl.ds(v * LANES, LANES)] = (pkv & 0xFFFF) + cN
                dloc[pl.ds(v * LANES, LANES)] = \
                    lax.shift_right_logical(pkv, 16)
            gat = pltpu.async_copy(h_hbm.at[sidx], hrows, sem)
            pltpu.sync_copy(e_hbm.at[pl.ds(ebase + t * CH, CH)], erows)
            gat.wait()
            for v in range(CH // LANES):
                dlv = dloc[pl.ds(v * LANES, LANES)]
                for j in range(LANES):
                    jj = v * LANES + j
                    dlb = dlv.at[jsel[j]].get(mode="promise_in_bounds")
                    v0 = jnp.maximum(
                        hrows[jj, pl.ds(0, LANES)]
                        + erows[jj, pl.ds(0, LANES)], 0.0)
                    v1 = jnp.maximum(
                        hrows[jj, pl.ds(LANES, LANES)]
                        + erows[jj, pl.ds(LANES, LANES)], 0.0)
                    plsc.addupdate_scatter(aggr, [dlb, iot], v0)
                    plsc.addupdate_scatter(aggr, [dlb, iot16], v1)
            return 0

        lax.fori_loop(0, nch, _chunk, 0)

    pltpu.sync_copy(aggr.at[pl.ds(0, BUCKET)],
                    out_hbm.at[pl.ds(c * NROWS + s * BUCKET, BUCKET)])


def _make_sc_aggr():
    mesh = plsc.VectorSubcoreMesh(core_axis_name="c", subcore_axis_name="s")
    return pl.kernel(
        _agg_body,
        out_type=jax.ShapeDtypeStruct((NC * NROWS, HH), jnp.float32),
        mesh=mesh,
        scratch_types=[
            pltpu.VMEM((CH,), jnp.int32),
            pltpu.VMEM((CH,), jnp.int32),
            pltpu.VMEM((CH,), jnp.int32),
            pltpu.VMEM((CH, HH), jnp.float32),
            pltpu.VMEM((CH, HH), jnp.float32),
            pltpu.VMEM((LANES,), jnp.int32),
            pltpu.VMEM((TROWS, HH), jnp.float32),
            pltpu.SemaphoreType.DMA,
        ],
        compiler_params=pltpu.CompilerParams(use_tc_tiling_on_sc=False, needs_layout_passes=False),
    )


_sc_aggr = _make_sc_aggr()


def _sc_layer(h2, eB, pkB, counts):
    h_flat = h2.reshape(2 * N, HH)
    e_flat = eB.reshape(2 * EA_ROWS, HH)
    out = _sc_aggr(h_flat, e_flat, pkB, counts)
    return out.reshape(2, NROWS, HH)[:, :N, :]


@jax.jit
def kernel(x, edge_index, edge_attr, batch, np_W, np_b,
           lin1_W, lin1_b, mlp1_W1, mlp1_b1, mlp1_W2, mlp1_b2, bn1_g, bn1_b,
           lin2_W, lin2_b, mlp2_W1, mlp2_b1, mlp2_W2, mlp2_b2, bn2_g, bn2_b,
           lin3_W, lin3_b, mlp3_W1, mlp3_b1, mlp3_W2, mlp3_b2, bn3_g, bn3_b,
           proj_W, proj_b, ln_g, ln_b):
    src = edge_index[0]
    dst = edge_index[1]
    srcp = jnp.pad(src, (0, EP - E))
    dstp = jnp.pad(dst, (0, EP - E), constant_values=TRASH)
    edge_attr_p = jnp.pad(edge_attr, ((0, EP - E), (0, 0)))

    pkB, ea_bucketed, counts = _sc_partition(srcp, dstp, edge_attr_p)

    w_all = jnp.concatenate([lin1_W, lin2_W, lin3_W], axis=1)
    b_all = jnp.concatenate([lin1_b, lin2_b, lin3_b], axis=0)
    e1, e2, e3 = _edge_lin(ea_bucketed, w_all, b_all)

    h = _node_proj(x, np_W, np_b)

    aggr = _sc_layer(h, e1, pkB, counts)
    h = _node_mlp(h, aggr, mlp1_W1, mlp1_b1, mlp1_W2, mlp1_b2, bn1_g, bn1_b)

    aggr = _sc_layer(h, e2, pkB, counts)
    h = _node_mlp(h, aggr, mlp2_W1, mlp2_b1, mlp2_W2, mlp2_b2, bn2_g, bn2_b)

    aggr = _sc_layer(h, e3, pkB, counts)
    h = _node_mlp(h, aggr, mlp3_W1, mlp3_b1, mlp3_W2, mlp3_b2, bn3_g, bn3_b)

    return _pool_proj(h, batch, proj_W, proj_b, ln_g, ln_b)


# speedup vs baseline: 1.8680x; 1.0472x over previous
import math

import jax
import jax.numpy as jnp
from jax import lax
from jax.experimental import pallas as pl
from jax.experimental.pallas import tpu as pltpu
from jax.experimental.pallas import tpu_sc as plsc

N = 50000
E = 800000
IN_DIM = 128
H = 64
HH = 32
ED = 16
EMB = 512
G = 64

NC = 2
NS = 16
LANES = 16

CH = 128
BLK = 256
EP = 800768
EPH = EP // 2
SCAN_CH = 1024
SCAN_NCH = EPH // SCAN_CH
BUCKET = 3200
CAPH = 28672
EA_ROWS = NC * NS * CAPH
TROWS = 3208
LTRASH = BUCKET
NROWS = NS * BUCKET
TRASH = N

_BN_SCALE = 1.0 / math.sqrt(1.0 + 1e-5)


def _nodeproj_body(x_ref, w_ref, b_ref, out_ref):
    r = jnp.dot(x_ref[...], w_ref[...], preferred_element_type=jnp.float32)
    r = r + b_ref[...]
    out_ref[0] = r[:, :HH]
    out_ref[1] = r[:, HH:]


def _node_proj(x, np_W, np_b):
    B = 2000
    nb = N // B
    return pl.pallas_call(
        _nodeproj_body,
        grid=(nb,),
        in_specs=[
            pl.BlockSpec((B, IN_DIM), lambda i: (i, 0)),
            pl.BlockSpec((IN_DIM, H), lambda i: (0, 0)),
            pl.BlockSpec((1, H), lambda i: (0, 0)),
        ],
        out_specs=pl.BlockSpec((2, B, HH), lambda i: (0, i, 0)),
        out_shape=jax.ShapeDtypeStruct((2, N, HH), jnp.float32),
    )(x, np_W, np_b.reshape(1, H))


def _edgelin_body(ea_ref, w_ref, b_ref, o1_ref, o2_ref, o3_ref):
    r = jnp.dot(ea_ref[...], w_ref[...], preferred_element_type=jnp.float32)
    r = r + b_ref[...]
    o1_ref[0] = r[:, 0:32]
    o1_ref[1] = r[:, 32:64]
    o2_ref[0] = r[:, 64:96]
    o2_ref[1] = r[:, 96:128]
    o3_ref[0] = r[:, 128:160]
    o3_ref[1] = r[:, 160:192]


def _edge_lin(ea_bucketed, w_all, b_all):
    B = 2048
    nb = EA_ROWS // B
    out_sds = jax.ShapeDtypeStruct((2, EA_ROWS, HH), jnp.float32)
    spec = pl.BlockSpec((2, B, HH), lambda i: (0, i, 0))
    return pl.pallas_call(
        _edgelin_body,
        grid=(nb,),
        in_specs=[
            pl.BlockSpec((B, ED), lambda i: (i, 0)),
            pl.BlockSpec((ED, 3 * H), lambda i: (0, 0)),
            pl.BlockSpec((1, 3 * H), lambda i: (0, 0)),
        ],
        out_specs=(spec, spec, spec),
        out_shape=(out_sds, out_sds, out_sds),
    )(ea_bucketed, w_all, b_all.reshape(1, 3 * H))


def _nodemlp_body(h_ref, a_ref, w1_ref, b1_ref, w2_ref, b2_ref, g_ref, bb_ref,
                  out_ref):
    hf = jnp.concatenate([h_ref[0], h_ref[1]], axis=1)
    af = jnp.concatenate([a_ref[0], a_ref[1]], axis=1)
    t = hf + af
    t = jnp.maximum(
        jnp.dot(t, w1_ref[...], preferred_element_type=jnp.float32)
        + b1_ref[...], 0.0)
    t = jnp.dot(t, w2_ref[...], preferred_element_type=jnp.float32) + b2_ref[...]
    t = t * (g_ref[...] * _BN_SCALE) + bb_ref[...]
    t = jnp.maximum(t, 0.0)
    out_ref[0] = t[:, :HH]
    out_ref[1] = t[:, HH:]


def _node_mlp(h2, aggr2, W1, b1, W2, b2, bn_g, bn_b):
    B = 2000
    nb = N // B
    spec = pl.BlockSpec((2, B, HH), lambda i: (0, i, 0))
    vec = lambda v: v.reshape(1, H)
    return pl.pallas_call(
        _nodemlp_body,
        grid=(nb,),
        in_specs=[
            spec, spec,
            pl.BlockSpec((H, H), lambda i: (0, 0)),
            pl.BlockSpec((1, H), lambda i: (0, 0)),
            pl.BlockSpec((H, H), lambda i: (0, 0)),
            pl.BlockSpec((1, H), lambda i: (0, 0)),
            pl.BlockSpec((1, H), lambda i: (0, 0)),
            pl.BlockSpec((1, H), lambda i: (0, 0)),
        ],
        out_specs=spec,
        out_shape=jax.ShapeDtypeStruct((2, N, HH), jnp.float32),
    )(h2, aggr2, W1, vec(b1), W2, vec(b2), vec(bn_g), vec(bn_b))


def _pool_body(h_ref, batch_ref, pw_ref, pb_ref, lg_ref, lb_ref, out_ref,
               acc_ref, mx_ref):
    i = pl.program_id(0)
    nb = pl.num_programs(0)

    @pl.when(i == 0)
    def _init():
        acc_ref[...] = jnp.zeros_like(acc_ref)
        mx_ref[...] = jnp.full_like(mx_ref, -jnp.inf)

    hf = jnp.concatenate([h_ref[0], h_ref[1]], axis=1)
    B = hf.shape[0]
    bb = batch_ref[0, 0]
    gid = lax.broadcasted_iota(jnp.int32, (1, G), 1)
    onehot = (bb[:, None] == gid).astype(jnp.float32)
    ones = jnp.ones((B, 1), jnp.float32)
    hx = jnp.concatenate([hf, ones, jnp.zeros((B, 63), jnp.float32)], axis=1)
    acc_ref[...] += jnp.dot(onehot.T, hx, preferred_element_type=jnp.float32)

    bbc = bb[:, None]
    parts = []
    for g in range(G):
        col = jnp.where(bbc == g, hf, -jnp.inf)
        parts.append(jnp.max(col, axis=0, keepdims=True))
    mx_ref[...] = jnp.maximum(mx_ref[...], jnp.concatenate(parts, axis=0))

    @pl.when(i == nb - 1)
    def _final():
        acc = acc_ref[...]
        sums = acc[:, :H]
        cnt = acc[:, H:H + 1]
        mean = sums / jnp.maximum(cnt, 1.0)
        cat = jnp.concatenate([mean, mx_ref[...]], axis=1)
        o = jnp.dot(cat, pw_ref[...], preferred_element_type=jnp.float32)
        o = o + pb_ref[...]
        mu = jnp.mean(o, axis=-1, keepdims=True)
        var = jnp.mean((o - mu) * (o - mu), axis=-1, keepdims=True)
        o = (o - mu) / jnp.sqrt(var + 1e-5) * lg_ref[...] + lb_ref[...]
        out_ref[...] = jnp.maximum(o, 0.0)


def _pool_proj(h2, batch, proj_W, proj_b, ln_g, ln_b):
    B = 1000
    nb = N // B
    batch_r = batch.reshape(nb, 1, B)
    return pl.pallas_call(
        _pool_body,
        grid=(nb,),
        in_specs=[
            pl.BlockSpec((2, B, HH), lambda i: (0, i, 0)),
            pl.BlockSpec((1, 1, B), lambda i: (i, 0, 0)),
            pl.BlockSpec((2 * H, EMB), lambda i: (0, 0)),
            pl.BlockSpec((1, EMB), lambda i: (0, 0)),
            pl.BlockSpec((1, EMB), lambda i: (0, 0)),
            pl.BlockSpec((1, EMB), lambda i: (0, 0)),
        ],
        out_specs=pl.BlockSpec((G, EMB), lambda i: (0, 0)),
        out_shape=jax.ShapeDtypeStruct((G, EMB), jnp.float32),
        scratch_shapes=[
            pltpu.VMEM((G, 2 * H), jnp.float32),
            pltpu.VMEM((G, H), jnp.float32),
        ],
    )(h2, batch_r, proj_W, proj_b.reshape(1, EMB), ln_g.reshape(1, EMB),
      ln_b.reshape(1, EMB))


def _part_body(src_hbm, dst_hbm, ea_hbm, pk_hbm, eab_hbm, cnt_hbm,
               sbuf, dbuf, pkbuf, eidbuf, earows, cntv, sem):
    c = lax.axis_index("c")
    s = lax.axis_index("s")
    iot = lax.broadcasted_iota(jnp.int32, (LANES,), 0)
    half_base = c * EPH
    s3200 = s * BUCKET

    def _scan_chunk(t, offv):
        base = half_base + t * SCAN_CH
        pltpu.sync_copy(src_hbm.at[pl.ds(base, SCAN_CH)], sbuf)
        pltpu.sync_copy(dst_hbm.at[pl.ds(base, SCAN_CH)], dbuf)

        def _vec(v, _):
            d = dbuf[pl.ds(v * LANES, LANES)]
            sv = sbuf[pl.ds(v * LANES, LANES)]
            b = lax.shift_right_logical(
                lax.shift_right_logical(d, 6) * 1311, 16)
            m = b == s
            mi = jnp.where(m, jnp.full((LANES,), 1, jnp.int32),
                           jnp.zeros((LANES,), jnp.int32))
            dloc = d - s3200
            pk = sv | lax.shift_left(dloc, 16)
            offv = cntv[...]
            pos = offv + plsc.cumsum(mi) - mi
            plsc.store_scatter(pkbuf, [pos], pk, mask=m)
            eid = (base + v * LANES) + iot
            plsc.store_scatter(eidbuf, [pos], eid, mask=m)
            cntv[...] = offv + plsc.all_reduce_population_count(m)
            return 0

        return lax.fori_loop(0, SCAN_CH // LANES, _vec, 0)

    cntv[...] = jnp.zeros((LANES,), jnp.int32)
    lax.fori_loop(0, SCAN_NCH, _scan_chunk, 0)
    offv = cntv[...]

    trash_pk = jnp.full((LANES,), LTRASH << 16, jnp.int32)
    zero16 = jnp.zeros((LANES,), jnp.int32)
    for j in range(BLK // LANES):
        pos = offv + iot + j * LANES
        plsc.store_scatter(pkbuf, [pos], trash_pk)
        plsc.store_scatter(eidbuf, [pos], zero16)

    row = c * NS + s
    pltpu.sync_copy(cntv, cnt_hbm.at[pl.ds(row * LANES, LANES)])
    pltpu.sync_copy(pkbuf, pk_hbm.at[pl.ds(row * CAPH, CAPH)])

    n = jnp.max(offv)
    nch = lax.shift_right_logical(n + (CH - 1), 7)

    def _gather_chunk(t, _):
        gat = pltpu.async_copy(
            ea_hbm.at[eidbuf.at[pl.ds(t * CH, CH)]], earows, sem)
        gat.wait()
        pltpu.sync_copy(earows, eab_hbm.at[pl.ds(row * CAPH + t * CH, CH)])
        return 0

    lax.fori_loop(0, nch, _gather_chunk, 0)


def _make_sc_partition():
    mesh = plsc.VectorSubcoreMesh(core_axis_name="c", subcore_axis_name="s")
    return pl.kernel(
        _part_body,
        out_type=(
            jax.ShapeDtypeStruct((NC * NS * CAPH,), jnp.int32),
            jax.ShapeDtypeStruct((EA_ROWS, ED), jnp.float32),
            jax.ShapeDtypeStruct((NC * NS * LANES,), jnp.int32),
        ),
        mesh=mesh,
        scratch_types=[
            pltpu.VMEM((SCAN_CH,), jnp.int32),
            pltpu.VMEM((SCAN_CH,), jnp.int32),
            pltpu.VMEM((CAPH,), jnp.int32),
            pltpu.VMEM((CAPH,), jnp.int32),
            pltpu.VMEM((CH, ED), jnp.float32),
            pltpu.VMEM((LANES,), jnp.int32),
            pltpu.SemaphoreType.DMA,
        ],
        compiler_params=pltpu.CompilerParams(use_tc_tiling_on_sc=False, needs_layout_passes=False),
    )


_sc_partition = _make_sc_partition()


def _agg_body(h_hbm, e_hbm, pk_hbm, cnt_hbm, out_hbm,
              pkbuf, sidx, dl32, hrows, erows, cntv, aggrf, sem):
    c = lax.axis_index("c")
    s = lax.axis_index("s")
    iot = lax.broadcasted_iota(jnp.int32, (LANES,), 0)
    iot16 = iot + LANES
    jsel = [jnp.full((LANES,), j, jnp.int32) for j in range(LANES)]
    zf = jnp.zeros((LANES,), jnp.float32)
    cN = c * N

    def _zrow(r, _):
        aggrf[pl.ds(r * LANES, LANES)] = zf
        return 0
    lax.fori_loop(0, TROWS * HH // LANES, _zrow, 0)

    def _segment(g, _):
        row = g * NS + s
        pltpu.sync_copy(cnt_hbm.at[pl.ds(row * LANES, LANES)], cntv)
        n = jnp.max(cntv[...])
        nblk = lax.shift_right_logical(n + (BLK - 1), 8)
        pkbase = row * CAPH
        ebase = c * EA_ROWS + row * CAPH

        def _block(t, _):
            pltpu.sync_copy(pk_hbm.at[pl.ds(pkbase + t * BLK, BLK)], pkbuf)
            for v in range(BLK // LANES):
                pkv = pkbuf[pl.ds(v * LANES, LANES)]
                sidx[pl.ds(v * LANES, LANES)] = (pkv & 0xFFFF) + cN
                dl32[pl.ds(v * LANES, LANES)] = \
                    lax.shift_right_logical(pkv & 0x7FFF0000, 11)
            gat0 = pltpu.async_copy(
                h_hbm.at[sidx.at[pl.ds(0, CH)]],
                hrows.at[pl.ds(0, CH)], sem)
            gat1 = pltpu.async_copy(
                h_hbm.at[sidx.at[pl.ds(CH, CH)]],
                hrows.at[pl.ds(CH, CH)], sem)
            pltpu.sync_copy(e_hbm.at[pl.ds(ebase + t * BLK, BLK)], erows)
            gat0.wait()
            gat1.wait()
            for v in range(BLK // LANES):
                dlv = dl32[pl.ds(v * LANES, LANES)]
                for j in range(LANES):
                    jj = v * LANES + j
                    dlb = dlv.at[jsel[j]].get(mode="promise_in_bounds")
                    a0 = dlb | iot
                    a1 = dlb | iot16
                    v0 = jnp.maximum(
                        hrows[jj, pl.ds(0, LANES)]
                        + erows[jj, pl.ds(0, LANES)], 0.0)
                    v1 = jnp.maximum(
                        hrows[jj, pl.ds(LANES, LANES)]
                        + erows[jj, pl.ds(LANES, LANES)], 0.0)
                    plsc.addupdate_scatter(aggrf, [a0], v0)
                    plsc.addupdate_scatter(aggrf, [a1], v1)
            return 0

        lax.fori_loop(0, nblk, _block, 0)
        return 0

    lax.fori_loop(0, NC, _segment, 0)

    pltpu.sync_copy(aggrf.at[pl.ds(0, BUCKET * HH)],
                    out_hbm.at[pl.ds((c * NS + s) * BUCKET * HH, BUCKET * HH)])


def _make_sc_aggr():
    mesh = plsc.VectorSubcoreMesh(core_axis_name="c", subcore_axis_name="s")
    return pl.kernel(
        _agg_body,
        out_type=jax.ShapeDtypeStruct((NC * NROWS * HH,), jnp.float32),
        mesh=mesh,
        scratch_types=[
            pltpu.VMEM((BLK,), jnp.int32),
            pltpu.VMEM((BLK,), jnp.int32),
            pltpu.VMEM((BLK,), jnp.int32),
            pltpu.VMEM((BLK, HH), jnp.float32),
            pltpu.VMEM((BLK, HH), jnp.float32),
            pltpu.VMEM((LANES,), jnp.int32),
            pltpu.VMEM((TROWS * HH,), jnp.float32),
            pltpu.SemaphoreType.DMA,
        ],
        compiler_params=pltpu.CompilerParams(use_tc_tiling_on_sc=False, needs_layout_passes=False),
    )


_sc_aggr = _make_sc_aggr()


def _sc_layer(h2, eB, pkB, counts):
    h_flat = h2.reshape(2 * N, HH)
    e_flat = eB.reshape(2 * EA_ROWS, HH)
    out = _sc_aggr(h_flat, e_flat, pkB, counts)
    return out.reshape(2, NROWS, HH)[:, :N, :]


@jax.jit
def kernel(x, edge_index, edge_attr, batch, np_W, np_b,
           lin1_W, lin1_b, mlp1_W1, mlp1_b1, mlp1_W2, mlp1_b2, bn1_g, bn1_b,
           lin2_W, lin2_b, mlp2_W1, mlp2_b1, mlp2_W2, mlp2_b2, bn2_g, bn2_b,
           lin3_W, lin3_b, mlp3_W1, mlp3_b1, mlp3_W2, mlp3_b2, bn3_g, bn3_b,
           proj_W, proj_b, ln_g, ln_b):
    src = edge_index[0]
    dst = edge_index[1]
    srcp = jnp.pad(src, (0, EP - E))
    dstp = jnp.pad(dst, (0, EP - E), constant_values=TRASH)
    edge_attr_p = jnp.pad(edge_attr, ((0, EP - E), (0, 0)))

    pkB, ea_bucketed, counts = _sc_partition(srcp, dstp, edge_attr_p)

    w_all = jnp.concatenate([lin1_W, lin2_W, lin3_W], axis=1)
    b_all = jnp.concatenate([lin1_b, lin2_b, lin3_b], axis=0)
    e1, e2, e3 = _edge_lin(ea_bucketed, w_all, b_all)

    h = _node_proj(x, np_W, np_b)

    aggr = _sc_layer(h, e1, pkB, counts)
    h = _node_mlp(h, aggr, mlp1_W1, mlp1_b1, mlp1_W2, mlp1_b2, bn1_g, bn1_b)

    aggr = _sc_layer(h, e2, pkB, counts)
    h = _node_mlp(h, aggr, mlp2_W1, mlp2_b1, mlp2_W2, mlp2_b2, bn2_g, bn2_b)

    aggr = _sc_layer(h, e3, pkB, counts)
    h = _node_mlp(h, aggr, mlp3_W1, mlp3_b1, mlp3_W2, mlp3_b2, bn3_g, bn3_b)

    return _pool_proj(h, batch, proj_W, proj_b, ln_g, ln_b)


# speedup vs baseline: 3.0768x; 1.6471x over previous
import math

import jax
import jax.numpy as jnp
from jax import lax
from jax.experimental import pallas as pl
from jax.experimental.pallas import tpu as pltpu
from jax.experimental.pallas import tpu_sc as plsc

N = 50000
E = 800000
IN_DIM = 128
H = 64
HH = 32
ED = 16
EMB = 512
G = 64

NC = 2
NS = 16
LANES = 16

CH = 128
BLK = 256
BPT = 196
EPT = BPT * BLK
EP = EPT * NS
NROWS = 51200
RPT = NROWS // NS
TRASH = N

_BN_SCALE = 1.0 / math.sqrt(1.0 + 1e-5)


def _nodeproj_body(x_ref, w_ref, b_ref, out_ref):
    r = jnp.dot(x_ref[...], w_ref[...], preferred_element_type=jnp.float32)
    r = r + b_ref[...]
    out_ref[0] = r[:, :HH]
    out_ref[1] = r[:, HH:]


def _node_proj(x, np_W, np_b):
    B = 2000
    nb = N // B
    return pl.pallas_call(
        _nodeproj_body,
        grid=(nb,),
        in_specs=[
            pl.BlockSpec((B, IN_DIM), lambda i: (i, 0)),
            pl.BlockSpec((IN_DIM, H), lambda i: (0, 0)),
            pl.BlockSpec((1, H), lambda i: (0, 0)),
        ],
        out_specs=pl.BlockSpec((2, B, HH), lambda i: (0, i, 0)),
        out_shape=jax.ShapeDtypeStruct((2, N, HH), jnp.float32),
    )(x, np_W, np_b.reshape(1, H))


def _edgelin_body(ea_ref, w_ref, b_ref, o1_ref, o2_ref, o3_ref):
    r = jnp.dot(ea_ref[...], w_ref[...], preferred_element_type=jnp.float32)
    r = r + b_ref[...]
    o1_ref[0] = r[:, 0:32]
    o1_ref[1] = r[:, 32:64]
    o2_ref[0] = r[:, 64:96]
    o2_ref[1] = r[:, 96:128]
    o3_ref[0] = r[:, 128:160]
    o3_ref[1] = r[:, 160:192]


def _edge_lin(edge_attr_p, w_all, b_all):
    B = 2048
    nb = EP // B
    out_sds = jax.ShapeDtypeStruct((2, EP, HH), jnp.float32)
    spec = pl.BlockSpec((2, B, HH), lambda i: (0, i, 0))
    return pl.pallas_call(
        _edgelin_body,
        grid=(nb,),
        in_specs=[
            pl.BlockSpec((B, ED), lambda i: (i, 0)),
            pl.BlockSpec((ED, 3 * H), lambda i: (0, 0)),
            pl.BlockSpec((1, 3 * H), lambda i: (0, 0)),
        ],
        out_specs=(spec, spec, spec),
        out_shape=(out_sds, out_sds, out_sds),
    )(edge_attr_p, w_all, b_all.reshape(1, 3 * H))


def _nodemlp_body(h_ref, a_ref, w1_ref, b1_ref, w2_ref, b2_ref, g_ref, bb_ref,
                  out_ref):
    hf = jnp.concatenate([h_ref[0], h_ref[1]], axis=1)
    af = jnp.concatenate([a_ref[0], a_ref[1]], axis=1)
    t = hf + af
    t = jnp.maximum(
        jnp.dot(t, w1_ref[...], preferred_element_type=jnp.float32)
        + b1_ref[...], 0.0)
    t = jnp.dot(t, w2_ref[...], preferred_element_type=jnp.float32) + b2_ref[...]
    t = t * (g_ref[...] * _BN_SCALE) + bb_ref[...]
    t = jnp.maximum(t, 0.0)
    out_ref[0] = t[:, :HH]
    out_ref[1] = t[:, HH:]


def _node_mlp(h2, aggr2, W1, b1, W2, b2, bn_g, bn_b):
    B = 2000
    nb = N // B
    spec = pl.BlockSpec((2, B, HH), lambda i: (0, i, 0))
    vec = lambda v: v.reshape(1, H)
    return pl.pallas_call(
        _nodemlp_body,
        grid=(nb,),
        in_specs=[
            spec, spec,
            pl.BlockSpec((H, H), lambda i: (0, 0)),
            pl.BlockSpec((1, H), lambda i: (0, 0)),
            pl.BlockSpec((H, H), lambda i: (0, 0)),
            pl.BlockSpec((1, H), lambda i: (0, 0)),
            pl.BlockSpec((1, H), lambda i: (0, 0)),
            pl.BlockSpec((1, H), lambda i: (0, 0)),
        ],
        out_specs=spec,
        out_shape=jax.ShapeDtypeStruct((2, N, HH), jnp.float32),
    )(h2, aggr2, W1, vec(b1), W2, vec(b2), vec(bn_g), vec(bn_b))


def _pool_body(h_ref, batch_ref, pw_ref, pb_ref, lg_ref, lb_ref, out_ref,
               acc_ref, mx_ref):
    i = pl.program_id(0)
    nb = pl.num_programs(0)

    @pl.when(i == 0)
    def _init():
        acc_ref[...] = jnp.zeros_like(acc_ref)
        mx_ref[...] = jnp.full_like(mx_ref, -jnp.inf)

    hf = jnp.concatenate([h_ref[0], h_ref[1]], axis=1)
    B = hf.shape[0]
    bb = batch_ref[0, 0]
    gid = lax.broadcasted_iota(jnp.int32, (1, G), 1)
    onehot = (bb[:, None] == gid).astype(jnp.float32)
    ones = jnp.ones((B, 1), jnp.float32)
    hx = jnp.concatenate([hf, ones, jnp.zeros((B, 63), jnp.float32)], axis=1)
    acc_ref[...] += jnp.dot(onehot.T, hx, preferred_element_type=jnp.float32)

    bbc = bb[:, None]
    parts = []
    for g in range(G):
        col = jnp.where(bbc == g, hf, -jnp.inf)
        parts.append(jnp.max(col, axis=0, keepdims=True))
    mx_ref[...] = jnp.maximum(mx_ref[...], jnp.concatenate(parts, axis=0))

    @pl.when(i == nb - 1)
    def _final():
        acc = acc_ref[...]
        sums = acc[:, :H]
        cnt = acc[:, H:H + 1]
        mean = sums / jnp.maximum(cnt, 1.0)
        cat = jnp.concatenate([mean, mx_ref[...]], axis=1)
        o = jnp.dot(cat, pw_ref[...], preferred_element_type=jnp.float32)
        o = o + pb_ref[...]
        mu = jnp.mean(o, axis=-1, keepdims=True)
        var = jnp.mean((o - mu) * (o - mu), axis=-1, keepdims=True)
        o = (o - mu) / jnp.sqrt(var + 1e-5) * lg_ref[...] + lb_ref[...]
        out_ref[...] = jnp.maximum(o, 0.0)


def _pool_proj(h2, batch, proj_W, proj_b, ln_g, ln_b):
    B = 1000
    nb = N // B
    batch_r = batch.reshape(nb, 1, B)
    return pl.pallas_call(
        _pool_body,
        grid=(nb,),
        in_specs=[
            pl.BlockSpec((2, B, HH), lambda i: (0, i, 0)),
            pl.BlockSpec((1, 1, B), lambda i: (i, 0, 0)),
            pl.BlockSpec((2 * H, EMB), lambda i: (0, 0)),
            pl.BlockSpec((1, EMB), lambda i: (0, 0)),
            pl.BlockSpec((1, EMB), lambda i: (0, 0)),
            pl.BlockSpec((1, EMB), lambda i: (0, 0)),
        ],
        out_specs=pl.BlockSpec((G, EMB), lambda i: (0, 0)),
        out_shape=jax.ShapeDtypeStruct((G, EMB), jnp.float32),
        scratch_shapes=[
            pltpu.VMEM((G, 2 * H), jnp.float32),
            pltpu.VMEM((G, H), jnp.float32),
        ],
    )(h2, batch_r, proj_W, proj_b.reshape(1, EMB), ln_g.reshape(1, EMB),
      ln_b.reshape(1, EMB))


def _sc_body(h_hbm, e_hbm, src_hbm, dst_hbm, out_hbm,
             sidx, didx, hrows, erows, aggr_sh, sem):
    c = lax.axis_index("c")
    s = lax.axis_index("s")

    def _zrow(r, _):
        hrows[r, pl.ds(0, LANES)] = jnp.zeros((LANES,), jnp.float32)
        hrows[r, pl.ds(LANES, LANES)] = jnp.zeros((LANES,), jnp.float32)
        return 0
    lax.fori_loop(0, CH, _zrow, 0)

    def _zchunk(z, _):
        pltpu.sync_copy(hrows.at[pl.ds(0, CH)],
                        aggr_sh.at[pl.ds(s * RPT + z * CH, CH)])
        return 0
    lax.fori_loop(0, RPT // CH, _zchunk, 0)

    plsc.subcore_barrier()

    coff = c * N
    eoff = c * (EP // CH)
    bbase = s * (EPT // CH)

    def _block(t, _):
        row0 = bbase + t * (BLK // CH)
        pltpu.sync_copy(src_hbm.at[pl.ds(row0, BLK // CH)], sidx)
        pltpu.sync_copy(dst_hbm.at[pl.ds(row0, BLK // CH)], didx)

        for rr in range(BLK // CH):
            for k in range(CH // LANES):
                sidx[rr, pl.ds(k * LANES, LANES)] = (
                    sidx[rr, pl.ds(k * LANES, LANES)] + coff)

        gat0 = pltpu.async_copy(h_hbm.at[sidx.at[0]],
                                hrows.at[pl.ds(0, CH)], sem)
        gat1 = pltpu.async_copy(h_hbm.at[sidx.at[1]],
                                hrows.at[pl.ds(CH, CH)], sem)
        pltpu.sync_copy(e_hbm.at[pl.ds((eoff + row0) * CH, BLK)], erows)
        gat0.wait()
        gat1.wait()

        def _row(r, _):
            a0 = hrows[r, pl.ds(0, LANES)] + erows[r, pl.ds(0, LANES)]
            a1 = hrows[r, pl.ds(LANES, LANES)] + erows[r, pl.ds(LANES, LANES)]
            hrows[r, pl.ds(0, LANES)] = jnp.maximum(a0, 0.0)
            hrows[r, pl.ds(LANES, LANES)] = jnp.maximum(a1, 0.0)
            return 0
        lax.fori_loop(0, BLK, _row, 0, unroll=4)

        pltpu.sync_copy(hrows.at[pl.ds(0, CH)], aggr_sh.at[didx.at[0]],
                        add=True)
        pltpu.sync_copy(hrows.at[pl.ds(CH, CH)], aggr_sh.at[didx.at[1]],
                        add=True)
        return 0

    lax.fori_loop(0, BPT, _block, 0)

    plsc.subcore_barrier()

    pltpu.sync_copy(aggr_sh.at[pl.ds(s * RPT, RPT)],
                    out_hbm.at[pl.ds(c * NROWS + s * RPT, RPT)])


def _make_sc_aggr():
    mesh = plsc.VectorSubcoreMesh(core_axis_name="c", subcore_axis_name="s")
    return pl.kernel(
        _sc_body,
        out_type=jax.ShapeDtypeStruct((NC * NROWS, HH), jnp.float32),
        mesh=mesh,
        scratch_types=[
            pltpu.VMEM((BLK // CH, CH), jnp.int32),
            pltpu.VMEM((BLK // CH, CH), jnp.int32),
            pltpu.VMEM((BLK, HH), jnp.float32),
            pltpu.VMEM((BLK, HH), jnp.float32),
            pltpu.VMEM_SHARED((NROWS, HH), jnp.float32),
            pltpu.SemaphoreType.DMA,
        ],
        compiler_params=pltpu.CompilerParams(use_tc_tiling_on_sc=False),
    )


_sc_aggr = _make_sc_aggr()


def _sc_layer(h2, e2, src2, dst2):
    h_flat = h2.reshape(2 * N, HH)
    e_flat = e2.reshape(2 * EP, HH)
    out = _sc_aggr(h_flat, e_flat, src2, dst2)
    return out.reshape(2, NROWS, HH)


@jax.jit
def kernel(x, edge_index, edge_attr, batch, np_W, np_b,
           lin1_W, lin1_b, mlp1_W1, mlp1_b1, mlp1_W2, mlp1_b2, bn1_g, bn1_b,
           lin2_W, lin2_b, mlp2_W1, mlp2_b1, mlp2_W2, mlp2_b2, bn2_g, bn2_b,
           lin3_W, lin3_b, mlp3_W1, mlp3_b1, mlp3_W2, mlp3_b2, bn3_g, bn3_b,
           proj_W, proj_b, ln_g, ln_b):
    src = edge_index[0]
    dst = edge_index[1]
    srcp = jnp.pad(src, (0, EP - E)).reshape(EP // CH, CH)
    dstp = jnp.pad(dst, (0, EP - E),
                   constant_values=TRASH).reshape(EP // CH, CH)
    edge_attr_p = jnp.pad(edge_attr, ((0, EP - E), (0, 0)))

    w_all = jnp.concatenate([lin1_W, lin2_W, lin3_W], axis=1)
    b_all = jnp.concatenate([lin1_b, lin2_b, lin3_b], axis=0)
    e1, e2, e3 = _edge_lin(edge_attr_p, w_all, b_all)

    h = _node_proj(x, np_W, np_b)

    aggr = _sc_layer(h, e1, srcp, dstp)
    h = _node_mlp(h, aggr, mlp1_W1, mlp1_b1, mlp1_W2, mlp1_b2, bn1_g, bn1_b)

    aggr = _sc_layer(h, e2, srcp, dstp)
    h = _node_mlp(h, aggr, mlp2_W1, mlp2_b1, mlp2_W2, mlp2_b2, bn2_g, bn2_b)

    aggr = _sc_layer(h, e3, srcp, dstp)
    h = _node_mlp(h, aggr, mlp3_W1, mlp3_b1, mlp3_W2, mlp3_b2, bn3_g, bn3_b)

    return _pool_proj(h, batch, proj_W, proj_b, ln_g, ln_b)


# speedup vs baseline: 3.0990x; 1.0072x over previous
import math

import jax
import jax.numpy as jnp
from jax import lax
from jax.experimental import pallas as pl
from jax.experimental.pallas import tpu as pltpu
from jax.experimental.pallas import tpu_sc as plsc

N = 50000
E = 800000
IN_DIM = 128
H = 64
HH = 32
ED = 16
EMB = 512
G = 64

NC = 2
NS = 16
LANES = 16

CH = 128
BLK = 256
BPT = 196
EPT = BPT * BLK
EP = EPT * NS
NROWS = 51200
RPT = NROWS // NS
TRASH = N

_BN_SCALE = 1.0 / math.sqrt(1.0 + 1e-5)


def _nodeproj_body(x_ref, w_ref, b_ref, out_ref):
    r = jnp.dot(x_ref[...], w_ref[...], preferred_element_type=jnp.float32)
    r = r + b_ref[...]
    out_ref[0] = r[:, :HH]
    out_ref[1] = r[:, HH:]


def _node_proj(x, np_W, np_b):
    B = 2000
    nb = N // B
    return pl.pallas_call(
        _nodeproj_body,
        grid=(nb,),
        in_specs=[
            pl.BlockSpec((B, IN_DIM), lambda i: (i, 0)),
            pl.BlockSpec((IN_DIM, H), lambda i: (0, 0)),
            pl.BlockSpec((1, H), lambda i: (0, 0)),
        ],
        out_specs=pl.BlockSpec((2, B, HH), lambda i: (0, i, 0)),
        out_shape=jax.ShapeDtypeStruct((2, N, HH), jnp.float32),
    )(x, np_W, np_b.reshape(1, H))


def _edgelin_body(ea_ref, w_ref, b_ref, o_ref):
    r = jnp.dot(ea_ref[...], w_ref[...], preferred_element_type=jnp.float32)
    r = r + b_ref[...]
    o_ref[0] = r[:, :HH]
    o_ref[1] = r[:, HH:]


def _edge_lin(edge_attr, lW, lb):
    B = 2000
    nb = E // B
    return pl.pallas_call(
        _edgelin_body,
        grid=(nb,),
        in_specs=[
            pl.BlockSpec((B, ED), lambda i: (i, 0)),
            pl.BlockSpec((ED, H), lambda i: (0, 0)),
            pl.BlockSpec((1, H), lambda i: (0, 0)),
        ],
        out_specs=pl.BlockSpec((2, B, HH), lambda i: (0, i, 0)),
        out_shape=jax.ShapeDtypeStruct((2, EP, HH), jnp.float32),
    )(edge_attr, lW, lb.reshape(1, H))


def _nodemlp_body(h_ref, a_ref, w1_ref, b1_ref, w2_ref, b2_ref, g_ref, bb_ref,
                  out_ref):
    hf = jnp.concatenate([h_ref[0], h_ref[1]], axis=1)
    af = jnp.concatenate([a_ref[0], a_ref[1]], axis=1)
    t = hf + af
    t = jnp.maximum(
        jnp.dot(t, w1_ref[...], preferred_element_type=jnp.float32)
        + b1_ref[...], 0.0)
    t = jnp.dot(t, w2_ref[...], preferred_element_type=jnp.float32) + b2_ref[...]
    t = t * (g_ref[...] * _BN_SCALE) + bb_ref[...]
    t = jnp.maximum(t, 0.0)
    out_ref[0] = t[:, :HH]
    out_ref[1] = t[:, HH:]


def _node_mlp(h2, aggr2, W1, b1, W2, b2, bn_g, bn_b):
    B = 2000
    nb = N // B
    spec = pl.BlockSpec((2, B, HH), lambda i: (0, i, 0))
    vec = lambda v: v.reshape(1, H)
    return pl.pallas_call(
        _nodemlp_body,
        grid=(nb,),
        in_specs=[
            spec, spec,
            pl.BlockSpec((H, H), lambda i: (0, 0)),
            pl.BlockSpec((1, H), lambda i: (0, 0)),
            pl.BlockSpec((H, H), lambda i: (0, 0)),
            pl.BlockSpec((1, H), lambda i: (0, 0)),
            pl.BlockSpec((1, H), lambda i: (0, 0)),
            pl.BlockSpec((1, H), lambda i: (0, 0)),
        ],
        out_specs=spec,
        out_shape=jax.ShapeDtypeStruct((2, N, HH), jnp.float32),
    )(h2, aggr2, W1, vec(b1), W2, vec(b2), vec(bn_g), vec(bn_b))


def _pool_body(h_ref, batch_ref, pw_ref, pb_ref, lg_ref, lb_ref, out_ref,
               acc_ref, mx_ref):
    i = pl.program_id(0)
    nb = pl.num_programs(0)

    @pl.when(i == 0)
    def _init():
        acc_ref[...] = jnp.zeros_like(acc_ref)
        mx_ref[...] = jnp.full_like(mx_ref, -jnp.inf)

    hf = jnp.concatenate([h_ref[0], h_ref[1]], axis=1)
    B = hf.shape[0]
    bb = batch_ref[0, 0]
    gid = lax.broadcasted_iota(jnp.int32, (1, G), 1)
    onehot = (bb[:, None] == gid).astype(jnp.float32)
    ones = jnp.ones((B, 1), jnp.float32)
    hx = jnp.concatenate([hf, ones, jnp.zeros((B, 63), jnp.float32)], axis=1)
    acc_ref[...] += jnp.dot(onehot.T, hx, preferred_element_type=jnp.float32)

    bbc = bb[:, None]
    parts = []
    for g in range(G):
        col = jnp.where(bbc == g, hf, -jnp.inf)
        parts.append(jnp.max(col, axis=0, keepdims=True))
    mx_ref[...] = jnp.maximum(mx_ref[...], jnp.concatenate(parts, axis=0))

    @pl.when(i == nb - 1)
    def _final():
        acc = acc_ref[...]
        sums = acc[:, :H]
        cnt = acc[:, H:H + 1]
        mean = sums / jnp.maximum(cnt, 1.0)
        cat = jnp.concatenate([mean, mx_ref[...]], axis=1)
        o = jnp.dot(cat, pw_ref[...], preferred_element_type=jnp.float32)
        o = o + pb_ref[...]
        mu = jnp.mean(o, axis=-1, keepdims=True)
        var = jnp.mean((o - mu) * (o - mu), axis=-1, keepdims=True)
        o = (o - mu) / jnp.sqrt(var + 1e-5) * lg_ref[...] + lb_ref[...]
        out_ref[...] = jnp.maximum(o, 0.0)


def _pool_proj(h2, batch, proj_W, proj_b, ln_g, ln_b):
    B = 1000
    nb = N // B
    batch_r = batch.reshape(nb, 1, B)
    return pl.pallas_call(
        _pool_body,
        grid=(nb,),
        in_specs=[
            pl.BlockSpec((2, B, HH), lambda i: (0, i, 0)),
            pl.BlockSpec((1, 1, B), lambda i: (i, 0, 0)),
            pl.BlockSpec((2 * H, EMB), lambda i: (0, 0)),
            pl.BlockSpec((1, EMB), lambda i: (0, 0)),
            pl.BlockSpec((1, EMB), lambda i: (0, 0)),
            pl.BlockSpec((1, EMB), lambda i: (0, 0)),
        ],
        out_specs=pl.BlockSpec((G, EMB), lambda i: (0, 0)),
        out_shape=jax.ShapeDtypeStruct((G, EMB), jnp.float32),
        scratch_shapes=[
            pltpu.VMEM((G, 2 * H), jnp.float32),
            pltpu.VMEM((G, H), jnp.float32),
        ],
    )(h2, batch_r, proj_W, proj_b.reshape(1, EMB), ln_g.reshape(1, EMB),
      ln_b.reshape(1, EMB))


def _sc_body(h_hbm, e_hbm, src_hbm, dst_hbm, out_hbm,
             sidx, didx, hrows, erows, aggr_sh, sem):
    c = lax.axis_index("c")
    s = lax.axis_index("s")

    def _zrow(r, _):
        hrows[r, pl.ds(0, LANES)] = jnp.zeros((LANES,), jnp.float32)
        hrows[r, pl.ds(LANES, LANES)] = jnp.zeros((LANES,), jnp.float32)
        return 0
    lax.fori_loop(0, CH, _zrow, 0)

    def _zchunk(z, _):
        pltpu.sync_copy(hrows.at[pl.ds(0, CH)],
                        aggr_sh.at[pl.ds(s * RPT + z * CH, CH)])
        return 0
    lax.fori_loop(0, RPT // CH, _zchunk, 0)

    plsc.subcore_barrier()

    coff = c * N
    eoff = c * (EP // CH)
    bbase = s * (EPT // CH)

    def _block(t, _):
        row0 = bbase + t * (BLK // CH)
        pltpu.sync_copy(src_hbm.at[pl.ds(row0, BLK // CH)], sidx)
        pltpu.sync_copy(dst_hbm.at[pl.ds(row0, BLK // CH)], didx)

        for rr in range(BLK // CH):
            for k in range(CH // LANES):
                sidx[rr, pl.ds(k * LANES, LANES)] = (
                    sidx[rr, pl.ds(k * LANES, LANES)] + coff)

        gat0 = pltpu.async_copy(h_hbm.at[sidx.at[0]],
                                hrows.at[pl.ds(0, CH)], sem)
        gat1 = pltpu.async_copy(h_hbm.at[sidx.at[1]],
                                hrows.at[pl.ds(CH, CH)], sem)
        pltpu.sync_copy(e_hbm.at[pl.ds((eoff + row0) * CH, BLK)], erows)
        gat0.wait()
        gat1.wait()

        def _row(r, _):
            a0 = hrows[r, pl.ds(0, LANES)] + erows[r, pl.ds(0, LANES)]
            a1 = hrows[r, pl.ds(LANES, LANES)] + erows[r, pl.ds(LANES, LANES)]
            hrows[r, pl.ds(0, LANES)] = jnp.maximum(a0, 0.0)
            hrows[r, pl.ds(LANES, LANES)] = jnp.maximum(a1, 0.0)
            return 0
        lax.fori_loop(0, BLK, _row, 0, unroll=4)

        pltpu.sync_copy(hrows.at[pl.ds(0, CH)], aggr_sh.at[didx.at[0]],
                        add=True)
        pltpu.sync_copy(hrows.at[pl.ds(CH, CH)], aggr_sh.at[didx.at[1]],
                        add=True)
        return 0

    lax.fori_loop(0, BPT, _block, 0)

    plsc.subcore_barrier()

    pltpu.sync_copy(aggr_sh.at[pl.ds(s * RPT, RPT)],
                    out_hbm.at[pl.ds(c * NROWS + s * RPT, RPT)])


def _make_sc_aggr():
    mesh = plsc.VectorSubcoreMesh(core_axis_name="c", subcore_axis_name="s")
    return pl.kernel(
        _sc_body,
        out_type=jax.ShapeDtypeStruct((NC * NROWS, HH), jnp.float32),
        mesh=mesh,
        scratch_types=[
            pltpu.VMEM((BLK // CH, CH), jnp.int32),
            pltpu.VMEM((BLK // CH, CH), jnp.int32),
            pltpu.VMEM((BLK, HH), jnp.float32),
            pltpu.VMEM((BLK, HH), jnp.float32),
            pltpu.VMEM_SHARED((NROWS, HH), jnp.float32),
            pltpu.SemaphoreType.DMA,
        ],
        compiler_params=pltpu.CompilerParams(use_tc_tiling_on_sc=False),
    )


_sc_aggr = _make_sc_aggr()


def _sc_layer(h2, e2, src2, dst2):
    h_flat = h2.reshape(2 * N, HH)
    e_flat = e2.reshape(2 * EP, HH)
    out = _sc_aggr(h_flat, e_flat, src2, dst2)
    return out.reshape(2, NROWS, HH)


@jax.jit
def kernel(x, edge_index, edge_attr, batch, np_W, np_b,
           lin1_W, lin1_b, mlp1_W1, mlp1_b1, mlp1_W2, mlp1_b2, bn1_g, bn1_b,
           lin2_W, lin2_b, mlp2_W1, mlp2_b1, mlp2_W2, mlp2_b2, bn2_g, bn2_b,
           lin3_W, lin3_b, mlp3_W1, mlp3_b1, mlp3_W2, mlp3_b2, bn3_g, bn3_b,
           proj_W, proj_b, ln_g, ln_b):
    src = edge_index[0]
    dst = edge_index[1]
    srcp = jnp.pad(src, (0, EP - E)).reshape(EP // CH, CH)
    dstp = jnp.pad(dst, (0, EP - E),
                   constant_values=TRASH).reshape(EP // CH, CH)

    e1 = _edge_lin(edge_attr, lin1_W, lin1_b)
    h = _node_proj(x, np_W, np_b)

    aggr = _sc_layer(h, e1, srcp, dstp)
    e2 = _edge_lin(edge_attr, lin2_W, lin2_b)
    h = _node_mlp(h, aggr, mlp1_W1, mlp1_b1, mlp1_W2, mlp1_b2, bn1_g, bn1_b)

    aggr = _sc_layer(h, e2, srcp, dstp)
    e3 = _edge_lin(edge_attr, lin3_W, lin3_b)
    h = _node_mlp(h, aggr, mlp2_W1, mlp2_b1, mlp2_W2, mlp2_b2, bn2_g, bn2_b)

    aggr = _sc_layer(h, e3, srcp, dstp)
    h = _node_mlp(h, aggr, mlp3_W1, mlp3_b1, mlp3_W2, mlp3_b2, bn3_g, bn3_b)

    return _pool_proj(h, batch, proj_W, proj_b, ln_g, ln_b)


# speedup vs baseline: 3.1001x; 1.0004x over previous
import math

import jax
import jax.numpy as jnp
from jax import lax
from jax.experimental import pallas as pl
from jax.experimental.pallas import tpu as pltpu
from jax.experimental.pallas import tpu_sc as plsc

N = 50000
E = 800000
IN_DIM = 128
H = 64
HH = 32
ED = 16
EMB = 512
G = 64

NC = 2
NS = 16
LANES = 16

CH = 128
BLK = 256
BPT = 196
EPT = BPT * BLK
EP = EPT * NS
NROWS = 51200
RPT = NROWS // NS
TRASH = N

_BN_SCALE = 1.0 / math.sqrt(1.0 + 1e-5)


def _nodeproj_body(x_ref, w_ref, b_ref, out_ref):
    r = jnp.dot(x_ref[...], w_ref[...], preferred_element_type=jnp.float32)
    r = r + b_ref[...]
    out_ref[0] = r[:, :HH]
    out_ref[1] = r[:, HH:]


def _node_proj(x, np_W, np_b):
    B = 2000
    nb = N // B
    return pl.pallas_call(
        _nodeproj_body,
        grid=(nb,),
        in_specs=[
            pl.BlockSpec((B, IN_DIM), lambda i: (i, 0)),
            pl.BlockSpec((IN_DIM, H), lambda i: (0, 0)),
            pl.BlockSpec((1, H), lambda i: (0, 0)),
        ],
        out_specs=pl.BlockSpec((2, B, HH), lambda i: (0, i, 0)),
        out_shape=jax.ShapeDtypeStruct((2, N, HH), jnp.float32),
    )(x, np_W, np_b.reshape(1, H))


def _edgelin_body(ea_ref, w_ref, b_ref, o_ref):
    r = jnp.dot(ea_ref[...], w_ref[...], preferred_element_type=jnp.float32)
    r = r + b_ref[...]
    o_ref[0] = r[:, :HH]
    o_ref[1] = r[:, HH:]


def _edge_lin(edge_attr, lW, lb):
    B = 2000
    nb = E // B
    return pl.pallas_call(
        _edgelin_body,
        grid=(nb,),
        in_specs=[
            pl.BlockSpec((B, ED), lambda i: (i, 0)),
            pl.BlockSpec((ED, H), lambda i: (0, 0)),
            pl.BlockSpec((1, H), lambda i: (0, 0)),
        ],
        out_specs=pl.BlockSpec((2, B, HH), lambda i: (0, i, 0)),
        out_shape=jax.ShapeDtypeStruct((2, EP, HH), jnp.float32),
    )(edge_attr, lW, lb.reshape(1, H))


def _nodemlp_body(h_ref, a_ref, w1_ref, b1_ref, w2_ref, b2_ref, g_ref, bb_ref,
                  out_ref):
    hf = jnp.concatenate([h_ref[0], h_ref[1]], axis=1)
    af = jnp.concatenate([a_ref[0], a_ref[1]], axis=1)
    t = hf + af
    t = jnp.maximum(
        jnp.dot(t, w1_ref[...], preferred_element_type=jnp.float32)
        + b1_ref[...], 0.0)
    t = jnp.dot(t, w2_ref[...], preferred_element_type=jnp.float32) + b2_ref[...]
    t = t * (g_ref[...] * _BN_SCALE) + bb_ref[...]
    t = jnp.maximum(t, 0.0)
    out_ref[0] = t[:, :HH]
    out_ref[1] = t[:, HH:]


def _node_mlp(h2, aggr2, W1, b1, W2, b2, bn_g, bn_b):
    B = 2000
    nb = N // B
    spec = pl.BlockSpec((2, B, HH), lambda i: (0, i, 0))
    vec = lambda v: v.reshape(1, H)
    return pl.pallas_call(
        _nodemlp_body,
        grid=(nb,),
        in_specs=[
            spec, spec,
            pl.BlockSpec((H, H), lambda i: (0, 0)),
            pl.BlockSpec((1, H), lambda i: (0, 0)),
            pl.BlockSpec((H, H), lambda i: (0, 0)),
            pl.BlockSpec((1, H), lambda i: (0, 0)),
            pl.BlockSpec((1, H), lambda i: (0, 0)),
            pl.BlockSpec((1, H), lambda i: (0, 0)),
        ],
        out_specs=spec,
        out_shape=jax.ShapeDtypeStruct((2, N, HH), jnp.float32),
    )(h2, aggr2, W1, vec(b1), W2, vec(b2), vec(bn_g), vec(bn_b))


def _pool_body(h_ref, batch_ref, pw_ref, pb_ref, lg_ref, lb_ref, out_ref,
               acc_ref, mx_ref):
    i = pl.program_id(0)
    nb = pl.num_programs(0)

    @pl.when(i == 0)
    def _init():
        acc_ref[...] = jnp.zeros_like(acc_ref)
        mx_ref[...] = jnp.full_like(mx_ref, -jnp.inf)

    hf = jnp.concatenate([h_ref[0], h_ref[1]], axis=1)
    B = hf.shape[0]
    bb = batch_ref[0, 0]
    gid = lax.broadcasted_iota(jnp.int32, (1, G), 1)
    onehot = (bb[:, None] == gid).astype(jnp.float32)
    ones = jnp.ones((B, 1), jnp.float32)
    hx = jnp.concatenate([hf, ones, jnp.zeros((B, 63), jnp.float32)], axis=1)
    acc_ref[...] += jnp.dot(onehot.T, hx, preferred_element_type=jnp.float32)

    bbc = bb[:, None]
    parts = []
    for g in range(G):
        col = jnp.where(bbc == g, hf, -jnp.inf)
        parts.append(jnp.max(col, axis=0, keepdims=True))
    mx_ref[...] = jnp.maximum(mx_ref[...], jnp.concatenate(parts, axis=0))

    @pl.when(i == nb - 1)
    def _final():
        acc = acc_ref[...]
        sums = acc[:, :H]
        cnt = acc[:, H:H + 1]
        mean = sums / jnp.maximum(cnt, 1.0)
        cat = jnp.concatenate([mean, mx_ref[...]], axis=1)
        o = jnp.dot(cat, pw_ref[...], preferred_element_type=jnp.float32)
        o = o + pb_ref[...]
        mu = jnp.mean(o, axis=-1, keepdims=True)
        var = jnp.mean((o - mu) * (o - mu), axis=-1, keepdims=True)
        o = (o - mu) / jnp.sqrt(var + 1e-5) * lg_ref[...] + lb_ref[...]
        out_ref[...] = jnp.maximum(o, 0.0)


def _pool_proj(h2, batch, proj_W, proj_b, ln_g, ln_b):
    B = 1000
    nb = N // B
    batch_r = batch.reshape(nb, 1, B)
    return pl.pallas_call(
        _pool_body,
        grid=(nb,),
        in_specs=[
            pl.BlockSpec((2, B, HH), lambda i: (0, i, 0)),
            pl.BlockSpec((1, 1, B), lambda i: (i, 0, 0)),
            pl.BlockSpec((2 * H, EMB), lambda i: (0, 0)),
            pl.BlockSpec((1, EMB), lambda i: (0, 0)),
            pl.BlockSpec((1, EMB), lambda i: (0, 0)),
            pl.BlockSpec((1, EMB), lambda i: (0, 0)),
        ],
        out_specs=pl.BlockSpec((G, EMB), lambda i: (0, 0)),
        out_shape=jax.ShapeDtypeStruct((G, EMB), jnp.float32),
        scratch_shapes=[
            pltpu.VMEM((G, 2 * H), jnp.float32),
            pltpu.VMEM((G, H), jnp.float32),
        ],
    )(h2, batch_r, proj_W, proj_b.reshape(1, EMB), ln_g.reshape(1, EMB),
      ln_b.reshape(1, EMB))


def _sc_body(h_hbm, e_hbm, src_hbm, dst_hbm, out_hbm,
             sidx, didx, hrows, erows, aggr_sh, sem):
    c = lax.axis_index("c")
    s = lax.axis_index("s")

    def _zrow(r, _):
        hrows[r, pl.ds(0, LANES)] = jnp.zeros((LANES,), jnp.float32)
        hrows[r, pl.ds(LANES, LANES)] = jnp.zeros((LANES,), jnp.float32)
        return 0
    lax.fori_loop(0, CH, _zrow, 0)

    def _zchunk(z, _):
        pltpu.sync_copy(hrows.at[pl.ds(0, CH)],
                        aggr_sh.at[pl.ds(s * RPT + z * CH, CH)])
        return 0
    lax.fori_loop(0, RPT // CH, _zchunk, 0)

    plsc.subcore_barrier()

    coff = c * N
    eoff = c * (EP // CH)
    bbase = s * (EPT // CH)

    def _block(t, _):
        row0 = bbase + t * (BLK // CH)
        pltpu.sync_copy(src_hbm.at[pl.ds(row0, BLK // CH)], sidx)
        pltpu.sync_copy(dst_hbm.at[pl.ds(row0, BLK // CH)], didx)

        for rr in range(BLK // CH):
            for k in range(CH // LANES):
                sidx[rr, pl.ds(k * LANES, LANES)] = (
                    sidx[rr, pl.ds(k * LANES, LANES)] + coff)

        gats = [
            pltpu.async_copy(h_hbm.at[sidx.at[rr]],
                             hrows.at[pl.ds(rr * CH, CH)], sem)
            for rr in range(BLK // CH)
        ]
        pltpu.sync_copy(e_hbm.at[pl.ds((eoff + row0) * CH, BLK)], erows)
        for gat in gats:
            gat.wait()

        def _row(r, _):
            a0 = hrows[r, pl.ds(0, LANES)] + erows[r, pl.ds(0, LANES)]
            a1 = hrows[r, pl.ds(LANES, LANES)] + erows[r, pl.ds(LANES, LANES)]
            hrows[r, pl.ds(0, LANES)] = jnp.maximum(a0, 0.0)
            hrows[r, pl.ds(LANES, LANES)] = jnp.maximum(a1, 0.0)
            return 0
        lax.fori_loop(0, BLK, _row, 0, unroll=4)

        for rr in range(BLK // CH):
            pltpu.sync_copy(hrows.at[pl.ds(rr * CH, CH)],
                            aggr_sh.at[didx.at[rr]], add=True)
        return 0

    lax.fori_loop(0, BPT, _block, 0)

    plsc.subcore_barrier()

    pltpu.sync_copy(aggr_sh.at[pl.ds(s * RPT, RPT)],
                    out_hbm.at[pl.ds(c * NROWS + s * RPT, RPT)])


def _make_sc_aggr():
    mesh = plsc.VectorSubcoreMesh(core_axis_name="c", subcore_axis_name="s")
    return pl.kernel(
        _sc_body,
        out_type=jax.ShapeDtypeStruct((NC * NROWS, HH), jnp.float32),
        mesh=mesh,
        scratch_types=[
            pltpu.VMEM((BLK // CH, CH), jnp.int32),
            pltpu.VMEM((BLK // CH, CH), jnp.int32),
            pltpu.VMEM((BLK, HH), jnp.float32),
            pltpu.VMEM((BLK, HH), jnp.float32),
            pltpu.VMEM_SHARED((NROWS, HH), jnp.float32),
            pltpu.SemaphoreType.DMA,
        ],
        compiler_params=pltpu.CompilerParams(use_tc_tiling_on_sc=False),
    )


_sc_aggr = _make_sc_aggr()


def _sc_layer(h2, e2, src2, dst2):
    h_flat = h2.reshape(2 * N, HH)
    e_flat = e2.reshape(2 * EP, HH)
    out = _sc_aggr(h_flat, e_flat, src2, dst2)
    return out.reshape(2, NROWS, HH)


@jax.jit
def kernel(x, edge_index, edge_attr, batch, np_W, np_b,
           lin1_W, lin1_b, mlp1_W1, mlp1_b1, mlp1_W2, mlp1_b2, bn1_g, bn1_b,
           lin2_W, lin2_b, mlp2_W1, mlp2_b1, mlp2_W2, mlp2_b2, bn2_g, bn2_b,
           lin3_W, lin3_b, mlp3_W1, mlp3_b1, mlp3_W2, mlp3_b2, bn3_g, bn3_b,
           proj_W, proj_b, ln_g, ln_b):
    src = edge_index[0]
    dst = edge_index[1]
    srcp = jnp.pad(src, (0, EP - E)).reshape(EP // CH, CH)
    dstp = jnp.pad(dst, (0, EP - E),
                   constant_values=TRASH).reshape(EP // CH, CH)

    e1 = _edge_lin(edge_attr, lin1_W, lin1_b)
    h = _node_proj(x, np_W, np_b)

    aggr = _sc_layer(h, e1, srcp, dstp)
    e2 = _edge_lin(edge_attr, lin2_W, lin2_b)
    h = _node_mlp(h, aggr, mlp1_W1, mlp1_b1, mlp1_W2, mlp1_b2, bn1_g, bn1_b)

    aggr = _sc_layer(h, e2, srcp, dstp)
    e3 = _edge_lin(edge_attr, lin3_W, lin3_b)
    h = _node_mlp(h, aggr, mlp2_W1, mlp2_b1, mlp2_W2, mlp2_b2, bn2_g, bn2_b)

    aggr = _sc_layer(h, e3, srcp, dstp)
    h = _node_mlp(h, aggr, mlp3_W1, mlp3_b1, mlp3_W2, mlp3_b2, bn3_g, bn3_b)

    return _pool_proj(h, batch, proj_W, proj_b, ln_g, ln_b)
